# baseline (speedup 1.0000x reference)
import functools

import jax
import jax.numpy as jnp
from jax import lax
from jax.experimental import pallas as pl
from jax.experimental.pallas import tpu as pltpu
from jax.experimental.pallas import tpu_sc as plsc

_B = 32
_S = 32
_V = 30522
_NT = 239
_L = 16


def _row_body(ids_hbm, w_hbm, out_hbm, ids_v, g_v, rowf, sem):
    wid = lax.axis_index("s") * 2 + lax.axis_index("c")

    pltpu.sync_copy(ids_hbm.at[wid], ids_v)
    gather = pltpu.async_copy(w_hbm.at[ids_v], g_v, sem)

    zero16 = jnp.zeros((_L,), jnp.float32)

    def _zero(j, carry):
        for c in range(8):
            rowf[j, pl.ds(c * _L, _L)] = zero16
        return carry

    lax.fori_loop(0, _NT, _zero, 0, unroll=2)

    gather.wait()

    i0 = ids_v[pl.ds(0, _L)]
    i1 = ids_v[pl.ds(_L, _L)]
    g0 = g_v[pl.ds(0, _L)]
    g1 = g_v[pl.ds(_L, _L)]

    first = lax.broadcast(i0[0], (_L,))
    diff = (i0 ^ first) | (i1 ^ first)
    acc = diff[0]
    for j in range(1, _L):
        acc = acc | diff[j]
    eqv = lax.broadcast(acc == 0, (_L,))
    v0 = jnp.where(eqv, g0, jnp.maximum(g0, 0.0))
    v1 = jnp.where(eqv, g1, jnp.maximum(g1, 0.0))

    plsc.store_scatter(rowf, [i0 >> 7, i0 & 127], v0)
    plsc.store_scatter(rowf, [i1 >> 7, i1 & 127], v1)

    pltpu.sync_copy(rowf, out_hbm.at[wid >> 3, :, wid & 7, :])


@jax.jit
def _encode(input_ids, weights):
    call = functools.partial(
        pl.kernel,
        out_type=jax.ShapeDtypeStruct((_B // 8, _NT, 8, 128), jnp.float32),
        mesh=plsc.VectorSubcoreMesh(core_axis_name="c", subcore_axis_name="s"),
        compiler_params=pltpu.CompilerParams(
            needs_layout_passes=False,
            use_tc_tiling_on_sc=False,
            disable_bounds_checks=True,
            skip_device_barrier=True,
        ),
        scratch_types=[
            pltpu.VMEM((_S,), jnp.int32),
            pltpu.VMEM((_S,), jnp.float32),
            pltpu.VMEM((_NT, 128), jnp.float32),
            pltpu.SemaphoreType.DMA,
        ],
    )(_row_body)
    return call(input_ids, weights)


def kernel(input_ids, weights):
    tiles = _encode(input_ids, weights)
    full = jnp.transpose(tiles, (0, 2, 1, 3)).reshape(_B, _NT * 128)
    return full[:, :_V].astype(jnp.bfloat16)


# speedup vs baseline: 1.0089x; 1.0089x over previous
import functools

import jax
import jax.numpy as jnp
from jax import lax
from jax.experimental import pallas as pl
from jax.experimental.pallas import tpu as pltpu
from jax.experimental.pallas import tpu_sc as plsc

_B = 32
_S = 32
_V = 30522
_NT = 239
_L = 16


def _row_body(ids_hbm, w_hbm, out_hbm, ids_v, g_v, rowf, sem):
    wid = lax.axis_index("s") * 2 + lax.axis_index("c")

    pltpu.sync_copy(ids_hbm.at[wid], ids_v)
    gather = pltpu.async_copy(w_hbm.at[ids_v], g_v, sem)

    zero16 = jnp.zeros((_L,), jnp.float32)

    def _zero(j, carry):
        for c in range(8):
            rowf[j, pl.ds(c * _L, _L)] = zero16
        return carry

    lax.fori_loop(0, 120, _zero, 0, unroll=2)

    gather.wait()

    i0 = ids_v[pl.ds(0, _L)]
    i1 = ids_v[pl.ds(_L, _L)]
    g0 = g_v[pl.ds(0, _L)]
    g1 = g_v[pl.ds(_L, _L)]

    first = lax.broadcast(i0[0], (_L,))
    diff = (i0 ^ first) | (i1 ^ first)
    acc = diff[0]
    for j in range(1, _L):
        acc = acc | diff[j]
    eqv = lax.broadcast(acc == 0, (_L,))
    v0 = jnp.where(eqv, g0, jnp.maximum(g0, 0.0))
    v1 = jnp.where(eqv, g1, jnp.maximum(g1, 0.0))

    j0 = i0 >> 7
    j1 = i1 >> 7
    m = wid >> 3
    r = wid & 7
    plsc.store_scatter(rowf, [j0, i0 & 127], v0, mask=j0 < 120)
    plsc.store_scatter(rowf, [j1, i1 & 127], v1, mask=j1 < 120)
    lo = pltpu.async_copy(
        rowf.at[pl.ds(0, 120), :], out_hbm.at[m, pl.ds(0, 120), r, :], sem
    )

    lax.fori_loop(120, _NT, _zero, 0, unroll=2)
    plsc.store_scatter(rowf, [j0, i0 & 127], v0, mask=j0 >= 120)
    plsc.store_scatter(rowf, [j1, i1 & 127], v1, mask=j1 >= 120)
    pltpu.sync_copy(
        rowf.at[pl.ds(120, _NT - 120), :],
        out_hbm.at[m, pl.ds(120, _NT - 120), r, :],
    )
    lo.wait()


@jax.jit
def _encode(input_ids, weights):
    call = functools.partial(
        pl.kernel,
        out_type=jax.ShapeDtypeStruct((_B // 8, _NT, 8, 128), jnp.float32),
        mesh=plsc.VectorSubcoreMesh(core_axis_name="c", subcore_axis_name="s"),
        compiler_params=pltpu.CompilerParams(
            needs_layout_passes=False, use_tc_tiling_on_sc=False
        ),
        scratch_types=[
            pltpu.VMEM((_S,), jnp.int32),
            pltpu.VMEM((_S,), jnp.float32),
            pltpu.VMEM((_NT, 128), jnp.float32),
            pltpu.SemaphoreType.DMA,
        ],
    )(_row_body)
    return call(input_ids, weights)


def kernel(input_ids, weights):
    tiles = _encode(input_ids, weights)
    full = jnp.transpose(tiles, (0, 2, 1, 3)).reshape(_B, _NT * 128)
    return full[:, :_V].astype(jnp.bfloat16)
